# baseline (speedup 1.0000x reference)
import functools

import jax
import jax.numpy as jnp
from jax import lax
from jax.experimental import pallas as pl
from jax.experimental.pallas import tpu as pltpu
from jax.experimental.pallas import tpu_sc as plsc

T = 9216
D = 64
V = 8192

TB = 1024
TSB = 128
N_TB = T // TB

NC, NS = 2, 16
NW = NC * NS
BPW = T // NW
CH = 96
NCH = BPW // CH


def _argmin_body(x_ref, e_ref, idx_ref, e2_ref):
    @pl.when(pl.program_id(0) == 0)
    def _():
        ef = e_ref[...]
        e2 = jnp.sum(ef * ef, axis=1)
        e2_ref[...] = e2.reshape(V // 128, 128)

    x = x_ref[...]
    raw = jax.lax.dot_general(
        x, e_ref[...], (((1,), (1,)), ((), ())),
        preferred_element_type=jnp.float32)
    lane = jax.lax.broadcasted_iota(jnp.int32, (TSB, 128), 1).astype(jnp.float32)
    for s in range(TB // TSB):
        best_val = jnp.full((TSB, 128), jnp.inf, dtype=jnp.float32)
        best_gid = jnp.zeros((TSB, 128), dtype=jnp.float32)
        rs = raw[s * TSB:(s + 1) * TSB, :]
        for G in range(V // 128):
            dg = -2.0 * rs[:, G * 128:(G + 1) * 128] + e2_ref[G][None, :]
            lt = dg < best_val
            best_gid = jnp.where(lt, jnp.float32(G), best_gid)
            best_val = jnp.minimum(dg, best_val)
        m = jnp.min(best_val, axis=1)
        cand = jnp.where(best_val == m[:, None],
                         best_gid * 128.0 + lane, jnp.float32(1e9))
        idx_ref[s] = jnp.min(cand, axis=1).astype(jnp.int32)


def _nearest_idx(xf, E):
    return pl.pallas_call(
        _argmin_body,
        grid=(N_TB,),
        in_specs=[
            pl.BlockSpec((TB, D), lambda i: (i, 0)),
            pl.BlockSpec((V, D), lambda i: (0, 0)),
        ],
        out_specs=pl.BlockSpec((TB // 128, 128), lambda i: (i, 0)),
        out_shape=jax.ShapeDtypeStruct((T // 128, 128), jnp.int32),
        scratch_shapes=[pltpu.VMEM((V // 128, 128), jnp.float32)],
    )(xf, E)


def _sc_gather(E, idx3d):
    mesh = plsc.VectorSubcoreMesh(core_axis_name="c", subcore_axis_name="s")

    @functools.partial(
        pl.kernel, mesh=mesh,
        compiler_params=pltpu.CompilerParams(use_tc_tiling_on_sc=False),
        out_type=jax.ShapeDtypeStruct((T, D), jnp.float32),
        scratch_types=[
            pltpu.VMEM((NCH, CH), jnp.int32),
            pltpu.VMEM((BPW, D), jnp.float32),
            pltpu.SemaphoreType.DMA,
        ],
    )
    def gather_k(table_hbm, idx_hbm, out_hbm, idx_v, rows_v, sem):
        wid = lax.axis_index("s") * NC + lax.axis_index("c")
        pltpu.sync_copy(idx_hbm.at[wid], idx_v)
        copies = []
        for j in range(NCH):
            copies.append(pltpu.async_copy(
                table_hbm.at[idx_v.at[j]],
                rows_v.at[pl.ds(j * CH, CH)], sem))
        for cp in copies:
            cp.wait()
        pltpu.sync_copy(rows_v, out_hbm.at[pl.ds(wid * BPW, BPW)])

    return gather_k(E, idx3d)


def kernel(x, E):
    batch_dim = x.shape[:-1]
    xf = x.reshape(-1, D)
    idx = _nearest_idx(xf, E)
    idx3d = idx.reshape(NW, NCH, CH)
    values = _sc_gather(E, idx3d)
    return values.reshape(*batch_dim, D)
